# baseline (speedup 1.0000x reference)
import jax
import jax.numpy as jnp
from jax.experimental import pallas as pl
from jax.experimental.pallas import tpu as pltpu

_ANG2BOHR = 1.8897259885789233
_HARTREE2EV = 27.211386245988
_A1 = 0.4289
_A2 = 4.4407
_S6 = 1.0
_S8 = 0.7875
_K1 = 16.0
_K3 = -4.0
_CUTOFF_BOHR = 22.0 * _ANG2BOHR
_NE = 128
_NR = 5
_F32 = jnp.float32
_BF16 = jnp.bfloat16
_DIMS_NN = (((1,), (0,)), ((), ()))
_DIMS_NT = (((1,), (1,)), ((), ()))


def _dot(a, b, dims):
    return jax.lax.dot_general(a, b, dims, preferred_element_type=_F32)


def _split(v):
    hi = v.astype(_BF16)
    lo = (v - hi.astype(_F32)).astype(_BF16)
    return hi, lo


def _d3_kernel(pos_ref, posr_ref, z_ref, zr_ref, rcov_ref, rcovc_ref,
               r4r2_ref, r4c_ref, cnref_ref, cnrefc_ref, t2_ref,
               e_ref, f_ref):
    for bb in range(pos_ref.shape[0]):
        _d3_one(bb, pos_ref, posr_ref, z_ref, zr_ref, rcov_ref, rcovc_ref,
                r4r2_ref, r4c_ref, cnref_ref, cnrefc_ref, t2_ref, e_ref, f_ref)


def _d3_one(bb, pos_ref, posr_ref, z_ref, zr_ref, rcov_ref, rcovc_ref,
            r4r2_ref, r4c_ref, cnref_ref, cnrefc_ref, t2_ref, e_ref, f_ref):
    n = pos_ref.shape[1]
    pos = pos_ref[bb]
    posr = posr_ref[bb]
    zc = z_ref[bb]
    zr = zr_ref[bb]

    oh = (jax.lax.broadcasted_iota(jnp.int32, (n, _NE), 1) == zc).astype(_F32)
    ohT = (jax.lax.broadcasted_iota(jnp.int32, (_NE, n), 0) == zr).astype(_F32)

    def sel(tab_row):
        return jnp.sum(oh * tab_row, axis=1, keepdims=True)

    def sel_row(tab_col):
        return jnp.sum(ohT * tab_col, axis=0, keepdims=True)

    def rowb(v):
        return jnp.transpose(jnp.broadcast_to(v, (n, n)))

    dx = pos[:, 0:1] - posr[0:1, :]
    dy = pos[:, 1:2] - posr[1:2, :]
    dz = pos[:, 2:3] - posr[2:3, :]
    ii = jax.lax.broadcasted_iota(jnp.int32, (n, n), 0)
    jj = jax.lax.broadcasted_iota(jnp.int32, (n, n), 1)
    eye = (ii == jj).astype(_F32)
    d2 = dx * dx + dy * dy + dz * dz + eye
    rinv = jax.lax.rsqrt(d2)
    r = d2 * rinv
    mask = (1.0 - eye) * (r <= _CUTOFF_BOHR).astype(_F32)

    rc_mat = sel(rcov_ref[...]) + sel_row(rcovc_ref[...])
    lg = 1.0 / (1.0 + jnp.exp(-_K1 * (rc_mat * rinv - 1.0)))
    mlg = mask * lg
    cn = jnp.sum(mlg, axis=1, keepdims=True)
    cn_row = jnp.sum(mlg, axis=0, keepdims=True)

    gs, hhs = [], []
    s_row = jnp.zeros((1, n), _F32)
    for a in range(_NR):
        t = cn - sel(cnref_ref[a:a + 1, :])
        ga = jnp.exp(_K3 * t * t)
        gs.append(ga)
        hhs.append((2.0 * _K3) * t * ga)
        tr = cn_row - sel_row(cnrefc_ref[:, a:a + 1])
        s_row = s_row + jnp.exp(_K3 * tr * tr)
    s = gs[0] + gs[1] + gs[2] + gs[3] + gs[4]
    hs = hhs[0] + hhs[1] + hhs[2] + hhs[3] + hhs[4]

    x = jnp.concatenate([ga * oh for ga in gs], axis=1)
    x_hi, x_lo = _split(x)
    t2_bf = t2_ref[...]
    w1 = _dot(x_hi, t2_bf, _DIMS_NN)
    num = _dot(w1.astype(_BF16), x_hi, _DIMS_NT)

    deni = 1.0 / (s * s_row + 1e-12)
    c6 = num * deni
    c6s = (num + jnp.transpose(num)) * deni

    qq = sel(r4r2_ref[...]) * sel_row(r4c_ref[...])
    r0 = _A1 * jnp.sqrt(3.0 * qq) + _A2
    r02 = r0 * r0
    r06 = r02 * r02 * r02
    r08 = r06 * r02
    r6 = d2 * d2 * d2
    r8 = r6 * d2
    d6i = 1.0 / (r6 + r06)
    d8i = 1.0 / (r8 + r08)
    w = -0.5 * mask * (_S6 * d6i + 3.0 * _S8 * qq * d8i)
    e_ref[bb] = jnp.sum(c6 * w, axis=(0, 1), keepdims=True) * _HARTREE2EV

    nb = w * deni
    dw2 = _dot(x_lo, t2_bf, _DIMS_NN)
    nb_hi, nb_lo = _split(nb)
    w1_hi, w1_lo = _split(w1)
    p2 = (_dot(nb_hi, w1_hi, _DIMS_NN) + _dot(nb_hi, w1_lo, _DIMS_NN)
          + _dot(nb_lo, w1_hi, _DIMS_NN))
    q2 = p2 + _dot(nb_hi, dw2.astype(_BF16), _DIMS_NN)
    sbar = -jnp.sum(nb * c6s * s_row, axis=1, keepdims=True)
    cnb = sbar * hs
    for a in range(_NR):
        gbar_a = jnp.sum((p2[:, a * _NE:(a + 1) * _NE]
                          + q2[:, a * _NE:(a + 1) * _NE]) * oh,
                         axis=1, keepdims=True)
        cnb = cnb + gbar_a * hhs[a]

    wp = mask * (1.5 * _S6 * d2 * d2 * d6i * d6i
                 + 6.0 * _S8 * qq * r6 * d8i * d8i)
    fp = (-0.5 * _K1) * mask * lg * (1.0 - lg) * rc_mat * (rinv * rinv * rinv)
    m = c6s * wp + (cnb + rowb(cnb)) * fp
    rs = jnp.sum(m, axis=1, keepdims=True)
    scale = -2.0 * _HARTREE2EV * _ANG2BOHR
    fcols = [scale * (pos[:, k:k + 1] * rs
                      - jnp.sum(m * posr[k:k + 1, :], axis=1, keepdims=True))
             for k in range(3)]
    f_ref[bb] = jnp.concatenate(fcols, axis=1)


def kernel(positions, cell, atomic_numbers, pbc, node_batch_index,
           covalent_radii, r4r2, c6_reference, coord_num_ref):
    b = cell.shape[0]
    n = positions.shape[0] // b
    nelem = covalent_radii.shape[0]

    pos_b = (positions.astype(_F32) * _ANG2BOHR).reshape(b, n, 3)
    pos_r = jnp.transpose(pos_b, (0, 2, 1))
    z = atomic_numbers.astype(jnp.int32).reshape(b, n, 1)
    zr = atomic_numbers.astype(jnp.int32).reshape(b, 1, n)

    rcov_t = jnp.zeros((1, _NE), _F32).at[0, :nelem].set(covalent_radii)
    rcov_c = jnp.zeros((_NE, 1), _F32).at[:nelem, 0].set(covalent_radii)
    r4_t = jnp.zeros((1, _NE), _F32).at[0, :nelem].set(r4r2)
    r4_c = jnp.zeros((_NE, 1), _F32).at[:nelem, 0].set(r4r2)
    cnref_t = jnp.zeros((_NR, _NE), _F32).at[:, :nelem].set(coord_num_ref.T)
    cnref_c = jnp.zeros((_NE, _NR), _F32).at[:nelem, :].set(coord_num_ref)
    t2 = jnp.zeros((_NR, _NE, _NR, _NE), _F32).at[:, :nelem, :, :nelem].set(
        jnp.transpose(c6_reference, (2, 0, 3, 1))).reshape(_NR * _NE, _NR * _NE)
    t2_bf = t2.astype(_BF16)

    e, f = pl.pallas_call(
        _d3_kernel,
        out_shape=[
            jax.ShapeDtypeStruct((b, 1, 1), _F32),
            jax.ShapeDtypeStruct((b, n, 3), _F32),
        ],
    )(pos_b, pos_r, z, zr, rcov_t, rcov_c, r4_t, r4_c, cnref_t, cnref_c, t2_bf)

    energy = e.reshape(b)
    forces = f.reshape(b * n, 3)
    volume = jnp.abs(jnp.linalg.det(cell))
    virial = jnp.einsum('bni,bnj->bij', positions.reshape(b, n, 3), f)
    stress = -virial / volume[:, None, None]
    s_sym = 0.5 * (stress + jnp.swapaxes(stress, 1, 2))
    stress_voigt = jnp.stack(
        [s_sym[:, 0, 0], s_sym[:, 1, 1], s_sym[:, 2, 2],
         s_sym[:, 1, 2], s_sym[:, 0, 2], s_sym[:, 0, 1]], axis=-1)
    return energy, forces, stress_voigt


# speedup vs baseline: 1.0714x; 1.0714x over previous
import jax
import jax.numpy as jnp
from jax.experimental import pallas as pl
from jax.experimental.pallas import tpu as pltpu

_ANG2BOHR = 1.8897259885789233
_HARTREE2EV = 27.211386245988
_A1 = 0.4289
_A2 = 4.4407
_S6 = 1.0
_S8 = 0.7875
_K1 = 16.0
_K3 = -4.0
_CUTOFF_BOHR = 22.0 * _ANG2BOHR
_NE = 128
_NR = 5
_F32 = jnp.float32
_BF16 = jnp.bfloat16
_DIMS_NN = (((1,), (0,)), ((), ()))
_DIMS_NT = (((1,), (1,)), ((), ()))


def _dot(a, b, dims):
    return jax.lax.dot_general(a, b, dims, preferred_element_type=_F32)


def _split(v):
    hi = v.astype(_BF16)
    lo = (v - hi.astype(_F32)).astype(_BF16)
    return hi, lo


def _d3_kernel(pos_ref, posr_ref, z_ref, zr_ref, rcov_ref, rcovc_ref,
               r4r2_ref, r4c_ref, cnref_ref, cnrefc_ref, t2_ref,
               e_ref, f_ref):
    bb = 0
    n = pos_ref.shape[1]
    pos = pos_ref[bb]
    posr = posr_ref[bb]
    zc = z_ref[bb]
    zr = zr_ref[bb]

    oh = (jax.lax.broadcasted_iota(jnp.int32, (n, _NE), 1) == zc).astype(_F32)
    ohT = (jax.lax.broadcasted_iota(jnp.int32, (_NE, n), 0) == zr).astype(_F32)

    def sel(tab_row):
        return jnp.sum(oh * tab_row, axis=1, keepdims=True)

    def sel_row(tab_col):
        return jnp.sum(ohT * tab_col, axis=0, keepdims=True)

    def rowb(v):
        return jnp.transpose(jnp.broadcast_to(v, (n, n)))

    dx = pos[:, 0:1] - posr[0:1, :]
    dy = pos[:, 1:2] - posr[1:2, :]
    dz = pos[:, 2:3] - posr[2:3, :]
    ii = jax.lax.broadcasted_iota(jnp.int32, (n, n), 0)
    jj = jax.lax.broadcasted_iota(jnp.int32, (n, n), 1)
    eye = (ii == jj).astype(_F32)
    d2 = dx * dx + dy * dy + dz * dz + eye
    r = jnp.sqrt(d2)
    rinv = 1.0 / r
    mask = (1.0 - eye) * (r <= _CUTOFF_BOHR).astype(_F32)

    rc_mat = sel(rcov_ref[...]) + sel_row(rcovc_ref[...])
    lg = 1.0 / (1.0 + jnp.exp(-_K1 * (rc_mat * rinv - 1.0)))
    mlg = mask * lg
    cn = jnp.sum(mlg, axis=1, keepdims=True)
    cn_row = jnp.sum(mlg, axis=0, keepdims=True)

    gs, hhs = [], []
    s_row = jnp.zeros((1, n), _F32)
    for a in range(_NR):
        t = cn - sel(cnref_ref[a:a + 1, :])
        ga = jnp.exp(_K3 * t * t)
        gs.append(ga)
        hhs.append((2.0 * _K3) * t * ga)
        tr = cn_row - sel_row(cnrefc_ref[:, a:a + 1])
        s_row = s_row + jnp.exp(_K3 * tr * tr)
    s = gs[0] + gs[1] + gs[2] + gs[3] + gs[4]
    hs = hhs[0] + hhs[1] + hhs[2] + hhs[3] + hhs[4]

    x = jnp.concatenate([ga * oh for ga in gs], axis=1)
    x_hi, x_lo = _split(x)
    t2_bf = t2_ref[...]
    w1 = _dot(x_hi, t2_bf, _DIMS_NN)
    num = _dot(w1.astype(_BF16), x_hi, _DIMS_NT)

    den = s * s_row + 1e-12
    c6 = num / den
    c6s = (num + jnp.transpose(num)) / den

    qq = sel(r4r2_ref[...]) * sel_row(r4c_ref[...])
    r0 = _A1 * jnp.sqrt(3.0 * qq) + _A2
    r02 = r0 * r0
    r06 = r02 * r02 * r02
    r08 = r06 * r02
    r6 = d2 * d2 * d2
    r8 = r6 * d2
    d6i = 1.0 / (r6 + r06)
    d8i = 1.0 / (r8 + r08)
    w = -0.5 * mask * (_S6 * d6i + 3.0 * _S8 * qq * d8i)
    e_ref[bb] = jnp.sum(c6 * w, axis=(0, 1), keepdims=True) * _HARTREE2EV

    nb = w / den
    dw2 = _dot(x_lo, t2_bf, _DIMS_NN)
    nb_hi, nb_lo = _split(nb)
    w1_hi, w1_lo = _split(w1)
    p2 = (_dot(nb_hi, w1_hi, _DIMS_NN) + _dot(nb_hi, w1_lo, _DIMS_NN)
          + _dot(nb_lo, w1_hi, _DIMS_NN))
    q2 = p2 + _dot(nb_hi, dw2.astype(_BF16), _DIMS_NN)
    sbar = -jnp.sum(nb * c6s * s_row, axis=1, keepdims=True)
    cnb = sbar * hs
    for a in range(_NR):
        gbar_a = jnp.sum((p2[:, a * _NE:(a + 1) * _NE]
                          + q2[:, a * _NE:(a + 1) * _NE]) * oh,
                         axis=1, keepdims=True)
        cnb = cnb + gbar_a * hhs[a]

    wp = mask * (1.5 * _S6 * d2 * d2 * d6i * d6i
                 + 6.0 * _S8 * qq * r6 * d8i * d8i)
    fp = (-0.5 * _K1) * mask * lg * (1.0 - lg) * rc_mat * (rinv * rinv * rinv)
    m = c6s * wp + (cnb + rowb(cnb)) * fp
    rs = jnp.sum(m, axis=1, keepdims=True)
    scale = -2.0 * _HARTREE2EV * _ANG2BOHR
    fcols = [scale * (pos[:, k:k + 1] * rs
                      - jnp.sum(m * posr[k:k + 1, :], axis=1, keepdims=True))
             for k in range(3)]
    f_ref[bb] = jnp.concatenate(fcols, axis=1)


def kernel(positions, cell, atomic_numbers, pbc, node_batch_index,
           covalent_radii, r4r2, c6_reference, coord_num_ref):
    b = cell.shape[0]
    n = positions.shape[0] // b
    nelem = covalent_radii.shape[0]

    pos_b = (positions.astype(_F32) * _ANG2BOHR).reshape(b, n, 3)
    pos_r = jnp.transpose(pos_b, (0, 2, 1))
    z = atomic_numbers.astype(jnp.int32).reshape(b, n, 1)
    zr = atomic_numbers.astype(jnp.int32).reshape(b, 1, n)

    rcov_t = jnp.zeros((1, _NE), _F32).at[0, :nelem].set(covalent_radii)
    rcov_c = jnp.zeros((_NE, 1), _F32).at[:nelem, 0].set(covalent_radii)
    r4_t = jnp.zeros((1, _NE), _F32).at[0, :nelem].set(r4r2)
    r4_c = jnp.zeros((_NE, 1), _F32).at[:nelem, 0].set(r4r2)
    cnref_t = jnp.zeros((_NR, _NE), _F32).at[:, :nelem].set(coord_num_ref.T)
    cnref_c = jnp.zeros((_NE, _NR), _F32).at[:nelem, :].set(coord_num_ref)
    t2 = jnp.zeros((_NR, _NE, _NR, _NE), _F32).at[:, :nelem, :, :nelem].set(
        jnp.transpose(c6_reference, (2, 0, 3, 1))).reshape(_NR * _NE, _NR * _NE)
    t2_bf = t2.astype(_BF16)

    e, f = pl.pallas_call(
        _d3_kernel,
        grid=(b,),
        in_specs=[
            pl.BlockSpec((1, n, 3), lambda i: (i, 0, 0)),
            pl.BlockSpec((1, 3, n), lambda i: (i, 0, 0)),
            pl.BlockSpec((1, n, 1), lambda i: (i, 0, 0)),
            pl.BlockSpec((1, 1, n), lambda i: (i, 0, 0)),
            pl.BlockSpec((1, _NE), lambda i: (0, 0)),
            pl.BlockSpec((_NE, 1), lambda i: (0, 0)),
            pl.BlockSpec((1, _NE), lambda i: (0, 0)),
            pl.BlockSpec((_NE, 1), lambda i: (0, 0)),
            pl.BlockSpec((_NR, _NE), lambda i: (0, 0)),
            pl.BlockSpec((_NE, _NR), lambda i: (0, 0)),
            pl.BlockSpec((_NR * _NE, _NR * _NE), lambda i: (0, 0)),
        ],
        out_specs=[
            pl.BlockSpec((1, 1, 1), lambda i: (i, 0, 0)),
            pl.BlockSpec((1, n, 3), lambda i: (i, 0, 0)),
        ],
        out_shape=[
            jax.ShapeDtypeStruct((b, 1, 1), _F32),
            jax.ShapeDtypeStruct((b, n, 3), _F32),
        ],
        compiler_params=pltpu.CompilerParams(
            dimension_semantics=("arbitrary",)),
    )(pos_b, pos_r, z, zr, rcov_t, rcov_c, r4_t, r4_c, cnref_t, cnref_c, t2_bf)

    energy = e.reshape(b)
    forces = f.reshape(b * n, 3)
    volume = jnp.abs(jnp.linalg.det(cell))
    virial = jnp.einsum('bni,bnj->bij', positions.reshape(b, n, 3), f)
    stress = -virial / volume[:, None, None]
    s_sym = 0.5 * (stress + jnp.swapaxes(stress, 1, 2))
    stress_voigt = jnp.stack(
        [s_sym[:, 0, 0], s_sym[:, 1, 1], s_sym[:, 2, 2],
         s_sym[:, 1, 2], s_sym[:, 0, 2], s_sym[:, 0, 1]], axis=-1)
    return energy, forces, stress_voigt


# speedup vs baseline: 1.1212x; 1.0465x over previous
import jax
import jax.numpy as jnp
from jax.experimental import pallas as pl
from jax.experimental.pallas import tpu as pltpu

_ANG2BOHR = 1.8897259885789233
_HARTREE2EV = 27.211386245988
_A1 = 0.4289
_A2 = 4.4407
_S6 = 1.0
_S8 = 0.7875
_K1 = 16.0
_K3 = -4.0
_CUTOFF_BOHR = 22.0 * _ANG2BOHR
_NE = 128
_NR = 5
_F32 = jnp.float32
_BF16 = jnp.bfloat16
_DIMS_NN = (((1,), (0,)), ((), ()))
_DIMS_NT = (((1,), (1,)), ((), ()))


def _dot(a, b, dims):
    return jax.lax.dot_general(a, b, dims, preferred_element_type=_F32)


def _split(v):
    hi = v.astype(_BF16)
    lo = (v - hi.astype(_F32)).astype(_BF16)
    return hi, lo


def _d3_kernel(pos_ref, posr_ref, z_ref, rcov_ref, r4r2_ref, cnref_ref,
               t2_ref, e_ref, f_ref):
    bb = 0
    n = pos_ref.shape[1]
    pos = pos_ref[bb]
    posr = posr_ref[bb]
    zc = z_ref[bb]

    oh = (jax.lax.broadcasted_iota(jnp.int32, (n, _NE), 1) == zc).astype(_F32)

    def sel(tab_row):
        return jnp.sum(oh * tab_row, axis=1, keepdims=True)

    def trow(v):
        return jnp.transpose(v)

    dx = pos[:, 0:1] - posr[0:1, :]
    dy = pos[:, 1:2] - posr[1:2, :]
    dz = pos[:, 2:3] - posr[2:3, :]
    ii = jax.lax.broadcasted_iota(jnp.int32, (n, n), 0)
    jj = jax.lax.broadcasted_iota(jnp.int32, (n, n), 1)
    eye = (ii == jj).astype(_F32)
    d2 = dx * dx + dy * dy + dz * dz + eye
    r = jnp.sqrt(d2)
    rinv = 1.0 / r
    mask = (1.0 - eye) * (r <= _CUTOFF_BOHR).astype(_F32)

    rcov_c = sel(rcov_ref[...])
    rc_mat = rcov_c + trow(rcov_c)
    lg = 1.0 / (1.0 + jnp.exp(-_K1 * (rc_mat * rinv - 1.0)))
    cn = jnp.sum(mask * lg, axis=1, keepdims=True)

    gs, hhs = [], []
    for a in range(_NR):
        t = cn - sel(cnref_ref[a:a + 1, :])
        ga = jnp.exp(_K3 * t * t)
        gs.append(ga)
        hhs.append((2.0 * _K3) * t * ga)
    s = gs[0] + gs[1] + gs[2] + gs[3] + gs[4]
    hs = hhs[0] + hhs[1] + hhs[2] + hhs[3] + hhs[4]
    s_row = trow(s)

    x = jnp.concatenate([ga * oh for ga in gs], axis=1)
    x_hi, x_lo = _split(x)
    t2_bf = t2_ref[...]
    w1 = _dot(x_hi, t2_bf, _DIMS_NN)
    num = _dot(w1.astype(_BF16), x_hi, _DIMS_NT)

    den = s * s_row + 1e-12
    c6 = num / den
    c6s = (num + jnp.transpose(num)) / den

    r4_c = sel(r4r2_ref[...])
    qq = r4_c * trow(r4_c)
    r0 = _A1 * jnp.sqrt(3.0 * qq) + _A2
    r02 = r0 * r0
    r06 = r02 * r02 * r02
    r08 = r06 * r02
    r6 = d2 * d2 * d2
    r8 = r6 * d2
    d6i = 1.0 / (r6 + r06)
    d8i = 1.0 / (r8 + r08)
    w = -0.5 * mask * (_S6 * d6i + 3.0 * _S8 * qq * d8i)
    e_ref[bb] = jnp.sum(c6 * w, axis=(0, 1), keepdims=True) * _HARTREE2EV

    nb = w / den
    dw2 = _dot(x_lo, t2_bf, _DIMS_NN)
    nb_hi, nb_lo = _split(nb)
    w1_hi, w1_lo = _split(w1)
    p2 = (_dot(nb_hi, w1_hi, _DIMS_NN) + _dot(nb_hi, w1_lo, _DIMS_NN)
          + _dot(nb_lo, w1_hi, _DIMS_NN))
    q2 = p2 + _dot(nb_hi, dw2.astype(_BF16), _DIMS_NN)
    sbar = -jnp.sum(nb * c6s * s_row, axis=1, keepdims=True)
    cnb = sbar * hs
    for a in range(_NR):
        gbar_a = jnp.sum((p2[:, a * _NE:(a + 1) * _NE]
                          + q2[:, a * _NE:(a + 1) * _NE]) * oh,
                         axis=1, keepdims=True)
        cnb = cnb + gbar_a * hhs[a]

    wp = mask * (1.5 * _S6 * d2 * d2 * d6i * d6i
                 + 6.0 * _S8 * qq * r6 * d8i * d8i)
    fp = (-0.5 * _K1) * mask * lg * (1.0 - lg) * rc_mat * (rinv * rinv * rinv)
    m = c6s * wp + (cnb + trow(cnb)) * fp
    rs = jnp.sum(m, axis=1, keepdims=True)
    scale = -2.0 * _HARTREE2EV * _ANG2BOHR
    fcols = [scale * (pos[:, k:k + 1] * rs
                      - jnp.sum(m * posr[k:k + 1, :], axis=1, keepdims=True))
             for k in range(3)]
    f_ref[bb] = jnp.concatenate(fcols, axis=1)


def kernel(positions, cell, atomic_numbers, pbc, node_batch_index,
           covalent_radii, r4r2, c6_reference, coord_num_ref):
    b = cell.shape[0]
    n = positions.shape[0] // b
    nelem = covalent_radii.shape[0]

    pos_b = (positions.astype(_F32) * _ANG2BOHR).reshape(b, n, 3)
    pos_r = jnp.transpose(pos_b, (0, 2, 1))
    z = atomic_numbers.astype(jnp.int32).reshape(b, n, 1)

    rcov_t = jnp.zeros((1, _NE), _F32).at[0, :nelem].set(covalent_radii)
    r4_t = jnp.zeros((1, _NE), _F32).at[0, :nelem].set(r4r2)
    cnref_t = jnp.zeros((_NR, _NE), _F32).at[:, :nelem].set(coord_num_ref.T)
    t2 = jnp.zeros((_NR, _NE, _NR, _NE), _F32).at[:, :nelem, :, :nelem].set(
        jnp.transpose(c6_reference, (2, 0, 3, 1))).reshape(_NR * _NE, _NR * _NE)
    t2_bf = t2.astype(_BF16)

    e, f = pl.pallas_call(
        _d3_kernel,
        grid=(b,),
        in_specs=[
            pl.BlockSpec((1, n, 3), lambda i: (i, 0, 0)),
            pl.BlockSpec((1, 3, n), lambda i: (i, 0, 0)),
            pl.BlockSpec((1, n, 1), lambda i: (i, 0, 0)),
            pl.BlockSpec((1, _NE), lambda i: (0, 0)),
            pl.BlockSpec((1, _NE), lambda i: (0, 0)),
            pl.BlockSpec((_NR, _NE), lambda i: (0, 0)),
            pl.BlockSpec((_NR * _NE, _NR * _NE), lambda i: (0, 0)),
        ],
        out_specs=[
            pl.BlockSpec((1, 1, 1), lambda i: (i, 0, 0)),
            pl.BlockSpec((1, n, 3), lambda i: (i, 0, 0)),
        ],
        out_shape=[
            jax.ShapeDtypeStruct((b, 1, 1), _F32),
            jax.ShapeDtypeStruct((b, n, 3), _F32),
        ],
        compiler_params=pltpu.CompilerParams(
            dimension_semantics=("arbitrary",)),
    )(pos_b, pos_r, z, rcov_t, r4_t, cnref_t, t2_bf)

    energy = e.reshape(b)
    forces = f.reshape(b * n, 3)
    volume = jnp.abs(jnp.linalg.det(cell))
    virial = jnp.einsum('bni,bnj->bij', positions.reshape(b, n, 3), f)
    stress = -virial / volume[:, None, None]
    s_sym = 0.5 * (stress + jnp.swapaxes(stress, 1, 2))
    stress_voigt = jnp.stack(
        [s_sym[:, 0, 0], s_sym[:, 1, 1], s_sym[:, 2, 2],
         s_sym[:, 1, 2], s_sym[:, 0, 2], s_sym[:, 0, 1]], axis=-1)
    return energy, forces, stress_voigt
